# Initial kernel scaffold; baseline (speedup 1.0000x reference)
import jax
import jax.numpy as jnp
from jax.experimental import pallas as pl


def kernel(x, edge_index, mlp_w, mlp_b, W_weight, W_bias, cls_w, cls_b):
    raise NotImplementedError("write your pallas kernel here")


# speedup vs baseline: 6.3085x; 6.3085x over previous
import functools

import jax
import jax.numpy as jnp
from jax import lax
from jax.experimental import pallas as pl
from jax.experimental.pallas import tpu as pltpu
from jax.experimental.pallas import tpu_sc as plsc

N = 10000
E = 320000
D = 128
OUT = 64
K = 4
A = 1.0
B = 1.0

NC = 2
NS = 16
NW = NC * NS
C = 120
NCHUNK = -(-E // C)
NCHUNK2 = -(-NCHUNK // NW) * NW
E2 = NCHUNK2 * C
NLOC = NCHUNK2 // NW
assert NLOC % 3 == 0 and NLOC % 2 == 0
N1 = 10240
ROWS_PER_SUB = N1 // NS
DEG_PER_SUB = N1 // NS
PAD_ROW = N1 - 1

_HI = jax.lax.Precision.HIGHEST

_COEF1 = (A - B) / 2.0
_COEF2 = (A + B + 2.0) / 2.0


def _phis(k):
    phi_k = (2 * k + A + B) * (2 * k + A + B - 1) / (2 * k * (k + A + B))
    phi_p = ((2 * k + A + B - 1) * (A ** 2 - B ** 2)
             / (2 * k * (k + A + B) * (2 * k + A + B - 2)))
    phi_pp = ((k + A - 1) * (k + B - 1) * (2 * k + A + B)
              / (k * (k + A + B) * (2 * k + A + B - 2)))
    return phi_k, phi_p, phi_pp


@functools.lru_cache(maxsize=None)
def _sc_mesh():
    return plsc.VectorSubcoreMesh(core_axis_name="c", subcore_axis_name="s",
                                  num_cores=NC, num_subcores=NS)


def _deg2_body(col_hbm, ones_hbm, zeros1_hbm, out_hbm,
               colbuf, ones_v, acc, sem):
    c = lax.axis_index("c")
    s = lax.axis_index("s")
    wid = s * NC + c

    pltpu.sync_copy(zeros1_hbm.at[pl.ds(0, DEG_PER_SUB)],
                    acc.at[pl.ds(s * DEG_PER_SUB, DEG_PER_SUB)])
    pltpu.sync_copy(ones_hbm, ones_v)
    plsc.subcore_barrier()

    def idx_start(jj, b):
        base = (wid + jj * NW) * C
        pltpu.async_copy(col_hbm.at[pl.ds(base, C)], colbuf.at[b], sem.at[b])

    def idx_wait(b):
        pltpu.make_async_copy(col_hbm.at[pl.ds(0, C)], colbuf.at[b],
                              sem.at[b]).wait()

    def scat(b):
        pltpu.sync_copy(ones_v, acc.at[colbuf.at[b]], add=True)

    idx_start(0, 0)

    def pair(p, _):
        for b in range(2):
            jj = 2 * p + 1 + b
            nb = (1 + b) % 2
            cb = b
            idx_start(jj, nb)
            idx_wait(cb)
            scat(cb)
        return 0

    lax.fori_loop(0, (NLOC - 2) // 2, pair, 0)
    idx_start(NLOC - 1, 1)
    idx_wait(0)
    scat(0)
    idx_wait(1)
    scat(1)
    plsc.subcore_barrier()

    pltpu.sync_copy(acc.at[pl.ds(s * DEG_PER_SUB, DEG_PER_SUB)],
                    out_hbm.at[c, pl.ds(s * DEG_PER_SUB, DEG_PER_SUB)])


@functools.lru_cache(maxsize=None)
def _deg2_kernel():
    return pl.kernel(
        _deg2_body,
        out_type=jax.ShapeDtypeStruct((NC, N1), jnp.float32),
        mesh=_sc_mesh(),
        scratch_types=[
            pltpu.VMEM((2, C), jnp.int32),
            pltpu.VMEM((C,), jnp.float32),
            pltpu.VMEM_SHARED((N1,), jnp.float32),
            pltpu.SemaphoreType.DMA((2,)),
        ],
    )


def _ppass_body(y_hbm, rc_hbm, zeros_hbm, out_hbm,
                idxbuf, rows, acc, semg, semi):
    c = lax.axis_index("c")
    s = lax.axis_index("s")
    wid = s * NC + c

    pltpu.sync_copy(zeros_hbm, acc.at[pl.ds(s * ROWS_PER_SUB, ROWS_PER_SUB)])
    plsc.subcore_barrier()

    def idx_start(jj, b):
        g = lax.min(wid + jj * NW, NCHUNK2 - 1)
        pltpu.async_copy(rc_hbm.at[g], idxbuf.at[b], semi.at[b])

    def idx_wait(b):
        pltpu.make_async_copy(rc_hbm.at[0], idxbuf.at[b], semi.at[b]).wait()

    def gather_start(b):
        pltpu.async_copy(y_hbm.at[idxbuf.at[b, 0]], rows.at[b], semg.at[b])

    def gather_wait(b):
        pltpu.make_async_copy(y_hbm.at[idxbuf.at[b, 0]], rows.at[b],
                              semg.at[b]).wait()

    def scat(b):
        pltpu.sync_copy(rows.at[b], acc.at[idxbuf.at[b, 1]], add=True)

    for j in range(2):
        idx_start(j, j)
    for j in range(2):
        idx_wait(j)
        gather_start(j)
    idx_start(2, 2)

    TRIPS = NLOC // 3 - 1

    def trip(p, _):
        for b in range(3):
            b2 = (b + 2) % 3
            jj = 3 * p + b
            idx_wait(b2)
            gather_start(b2)
            gather_wait(b)
            scat(b)
            idx_start(jj + 3, b)
        return 0

    lax.fori_loop(0, TRIPS, trip, 0)
    idx_wait((NLOC - 1) % 3)
    gather_start((NLOC - 1) % 3)
    for j in range(NLOC - 3, NLOC):
        gather_wait(j % 3)
        scat(j % 3)
    plsc.subcore_barrier()

    pltpu.sync_copy(acc.at[pl.ds(s * ROWS_PER_SUB, ROWS_PER_SUB)],
                    out_hbm.at[c, pl.ds(s * ROWS_PER_SUB, ROWS_PER_SUB)])


@functools.lru_cache(maxsize=None)
def _ppass_kernel():
    return pl.kernel(
        _ppass_body,
        out_type=jax.ShapeDtypeStruct((NC, N1, D), jnp.float32),
        mesh=_sc_mesh(),
        scratch_types=[
            pltpu.VMEM((3, 2, C), jnp.int32),
            pltpu.VMEM((3, C, D), jnp.float32),
            pltpu.VMEM_SHARED((N1, D), jnp.float32),
            pltpu.SemaphoreType.DMA((3,)),
            pltpu.SemaphoreType.DMA((3,)),
        ],
    )


BN = 1000
GRID = N // BN


def _mlp_body(x_ref, wT_ref, b_ref, h_ref):
    h = jnp.dot(x_ref[...], wT_ref[...], precision=_HI,
                preferred_element_type=jnp.float32)
    h_ref[...] = jnp.maximum(h + b_ref[...], 0.0)


def _mlp(x, mlp_wT, mlp_b2):
    return pl.pallas_call(
        _mlp_body,
        grid=(GRID,),
        in_specs=[
            pl.BlockSpec((BN, D), lambda i: (i, 0)),
            pl.BlockSpec((D, D), lambda i: (0, 0)),
            pl.BlockSpec((1, D), lambda i: (0, 0)),
        ],
        out_specs=pl.BlockSpec((BN, D), lambda i: (i, 0)),
        out_shape=jax.ShapeDtypeStruct((N, D), jnp.float32),
    )(x, mlp_wT, mlp_b2)


def _scale0_body(deg_ref, h_ref, dinv_ref, y_ref):
    deg = deg_ref[0] + deg_ref[1]
    dinv = jnp.where(deg > 0, lax.rsqrt(jnp.maximum(deg, 1e-12)), 0.0)
    dinv_ref[...] = dinv
    y_ref[...] = h_ref[...] * dinv


def _scale0(deg_col, h):
    return pl.pallas_call(
        _scale0_body,
        grid=(GRID,),
        in_specs=[
            pl.BlockSpec((NC, BN, 1), lambda i: (0, i, 0)),
            pl.BlockSpec((BN, D), lambda i: (i, 0)),
        ],
        out_specs=[
            pl.BlockSpec((BN, 1), lambda i: (i, 0)),
            pl.BlockSpec((BN, D), lambda i: (i, 0)),
        ],
        out_shape=[
            jax.ShapeDtypeStruct((N, 1), jnp.float32),
            jax.ShapeDtypeStruct((N, D), jnp.float32),
        ],
    )(deg_col, h)


def _comb_body(part_ref, dinv_ref, zlast_ref, zprev_ref, z_ref, y_ref,
               *, ca, cb, cc):
    dinv = dinv_ref[...]
    s = (part_ref[0] + part_ref[1]) * dinv
    z = ca * s + cb * zlast_ref[...] + cc * zprev_ref[...]
    z_ref[...] = z
    y_ref[...] = z * dinv


def _comb(part, dinv_col, z_last, z_prev, ca, cb, cc):
    return pl.pallas_call(
        functools.partial(_comb_body, ca=ca, cb=cb, cc=cc),
        grid=(GRID,),
        in_specs=[
            pl.BlockSpec((NC, BN, D), lambda i: (0, i, 0)),
            pl.BlockSpec((BN, 1), lambda i: (i, 0)),
            pl.BlockSpec((BN, D), lambda i: (i, 0)),
            pl.BlockSpec((BN, D), lambda i: (i, 0)),
        ],
        out_specs=[
            pl.BlockSpec((BN, D), lambda i: (i, 0)),
            pl.BlockSpec((BN, D), lambda i: (i, 0)),
        ],
        out_shape=[
            jax.ShapeDtypeStruct((N, D), jnp.float32),
            jax.ShapeDtypeStruct((N, D), jnp.float32),
        ],
    )(part, dinv_col, z_last, z_prev)


def _q_body(zs_ref, w_ref, b_ref, q_ref):
    i = pl.program_id(0)

    @pl.when(i == 0)
    def _():
        q_ref[...] = jnp.zeros_like(q_ref)

    q_ref[...] += jnp.sum(zs_ref[...], axis=1)

    @pl.when(i == GRID - 1)
    def _():
        zbar = q_ref[...] / float(N)
        rows = [
            jnp.dot(zbar[k:k + 1, :], w_ref[k], precision=_HI,
                    preferred_element_type=jnp.float32) + b_ref[k:k + 1, :]
            for k in range(K + 1)
        ]
        q_ref[...] = jnp.concatenate(rows, axis=0)


def _q_kernel(zs, w, b):
    return pl.pallas_call(
        _q_body,
        grid=(GRID,),
        in_specs=[
            pl.BlockSpec((K + 1, BN, D), lambda i: (0, i, 0)),
            pl.BlockSpec((K + 1, D, D), lambda i: (0, 0, 0)),
            pl.BlockSpec((K + 1, D), lambda i: (0, 0)),
        ],
        out_specs=pl.BlockSpec((K + 1, D), lambda i: (0, 0)),
        out_shape=jax.ShapeDtypeStruct((K + 1, D), jnp.float32),
    )(zs, w, b)


def _final_body(zs_ref, w_ref, b_ref, q_ref, clsT_ref, clsb_ref,
                out_ref, zt_ref, alpha_ref):
    hs = [
        jnp.dot(zs_ref[k], w_ref[k], precision=_HI,
                preferred_element_type=jnp.float32) + b_ref[k:k + 1, :]
        for k in range(K + 1)
    ]
    scores = jnp.concatenate(
        [jnp.sum(hs[k] * q_ref[k:k + 1, :], axis=1, keepdims=True)
         for k in range(K + 1)], axis=1)
    scores = jnp.tanh(scores)
    m = jnp.max(scores, axis=1, keepdims=True)
    ex = jnp.exp(scores - m)
    alpha = ex / jnp.sum(ex, axis=1, keepdims=True)
    alpha_ref[...] = alpha
    zt = alpha[:, 0:1] * hs[0]
    for k in range(1, K + 1):
        zt = zt + alpha[:, k:k + 1] * hs[k]
    zt = jnp.maximum(zt, 0.0)
    zt_ref[...] = zt
    out_ref[...] = jnp.dot(zt, clsT_ref[...], precision=_HI,
                           preferred_element_type=jnp.float32) + clsb_ref[...]


def _final(zs, w, b, q, cls_wT, cls_b2):
    return pl.pallas_call(
        _final_body,
        grid=(GRID,),
        in_specs=[
            pl.BlockSpec((K + 1, BN, D), lambda i: (0, i, 0)),
            pl.BlockSpec((K + 1, D, D), lambda i: (0, 0, 0)),
            pl.BlockSpec((K + 1, D), lambda i: (0, 0)),
            pl.BlockSpec((K + 1, D), lambda i: (0, 0)),
            pl.BlockSpec((D, OUT), lambda i: (0, 0)),
            pl.BlockSpec((1, OUT), lambda i: (0, 0)),
        ],
        out_specs=[
            pl.BlockSpec((BN, OUT), lambda i: (i, 0)),
            pl.BlockSpec((BN, D), lambda i: (i, 0)),
            pl.BlockSpec((BN, K + 1), lambda i: (i, 0)),
        ],
        out_shape=[
            jax.ShapeDtypeStruct((N, OUT), jnp.float32),
            jax.ShapeDtypeStruct((N, D), jnp.float32),
            jax.ShapeDtypeStruct((N, K + 1), jnp.float32),
        ],
    )(zs, w, b, q, cls_wT, cls_b2)


def kernel(x, edge_index, mlp_w, mlp_b, W_weight, W_bias, cls_w, cls_b):
    row = edge_index[0].astype(jnp.int32)
    col = edge_index[1].astype(jnp.int32)
    col, row = lax.sort_key_val(col, row)

    pad = E2 - E
    padfill = jnp.full((pad,), PAD_ROW, jnp.int32)
    colp = jnp.concatenate([col, jnp.zeros((pad,), jnp.int32)])
    rowp = jnp.concatenate([row, padfill])
    rc = jnp.stack([colp.reshape(NCHUNK2, C), rowp.reshape(NCHUNK2, C)],
                   axis=1)
    col_deg = jnp.concatenate([col, padfill])

    ones_c = jnp.ones((C,), jnp.float32)
    zeros1 = jnp.zeros((DEG_PER_SUB,), jnp.float32)
    zeros2 = jnp.zeros((ROWS_PER_SUB, D), jnp.float32)

    h = _mlp(x, mlp_w.T, mlp_b.reshape(1, D))

    deg2 = _deg2_kernel()(col_deg, ones_c, zeros1)
    deg_col = deg2[:, :N].reshape(NC, N, 1)

    dinv_col, y = _scale0(deg_col, h)

    z_list = [h]
    part = _ppass_kernel()(y, rc, zeros2)
    z, y = _comb(part, dinv_col, h, h, _COEF2, _COEF1, 0.0)
    z_list.append(z)
    for k in range(2, K + 1):
        phi_k, phi_p, phi_pp = _phis(k)
        part = _ppass_kernel()(y, rc, zeros2)
        z, y = _comb(part, dinv_col, z_list[-1], z_list[-2],
                     phi_k, phi_p, -phi_pp)
        z_list.append(z)

    zs = jnp.stack(z_list, axis=0)
    q = _q_kernel(zs, W_weight, W_bias)
    out, zt, alpha = _final(zs, W_weight, W_bias, q, cls_w.T,
                            cls_b.reshape(1, OUT))
    return (out, zt, zs, alpha)


# speedup vs baseline: 6.3196x; 1.0018x over previous
import functools

import jax
import jax.numpy as jnp
from jax import lax
from jax.experimental import pallas as pl
from jax.experimental.pallas import tpu as pltpu
from jax.experimental.pallas import tpu_sc as plsc

N = 10000
E = 320000
D = 128
OUT = 64
K = 4
A = 1.0
B = 1.0

NC = 2
NS = 16
NW = NC * NS
C = 88
NBUF = 4
NCHUNK = -(-E // C)
NCHUNK2 = -(-NCHUNK // (4 * NW)) * (4 * NW)
E2 = NCHUNK2 * C
NLOC = NCHUNK2 // NW
assert NLOC % 4 == 0 and NLOC % 2 == 0
N1 = 10240
ROWS_PER_SUB = N1 // NS
DEG_PER_SUB = N1 // NS
PAD_ROW = N1 - 1

_HI = jax.lax.Precision.HIGHEST

_COEF1 = (A - B) / 2.0
_COEF2 = (A + B + 2.0) / 2.0


def _phis(k):
    phi_k = (2 * k + A + B) * (2 * k + A + B - 1) / (2 * k * (k + A + B))
    phi_p = ((2 * k + A + B - 1) * (A ** 2 - B ** 2)
             / (2 * k * (k + A + B) * (2 * k + A + B - 2)))
    phi_pp = ((k + A - 1) * (k + B - 1) * (2 * k + A + B)
              / (k * (k + A + B) * (2 * k + A + B - 2)))
    return phi_k, phi_p, phi_pp


@functools.lru_cache(maxsize=None)
def _sc_mesh():
    return plsc.VectorSubcoreMesh(core_axis_name="c", subcore_axis_name="s",
                                  num_cores=NC, num_subcores=NS)


def _deg2_body(col_hbm, ones_hbm, zeros1_hbm, out_hbm,
               colbuf, ones_v, acc, sem):
    c = lax.axis_index("c")
    s = lax.axis_index("s")
    wid = s * NC + c

    pltpu.sync_copy(zeros1_hbm.at[pl.ds(0, DEG_PER_SUB)],
                    acc.at[pl.ds(s * DEG_PER_SUB, DEG_PER_SUB)])
    pltpu.sync_copy(ones_hbm, ones_v)
    plsc.subcore_barrier()

    def idx_start(jj, b):
        base = (wid + jj * NW) * C
        pltpu.async_copy(col_hbm.at[pl.ds(base, C)], colbuf.at[b], sem.at[b])

    def idx_wait(b):
        pltpu.make_async_copy(col_hbm.at[pl.ds(0, C)], colbuf.at[b],
                              sem.at[b]).wait()

    def scat(b):
        pltpu.sync_copy(ones_v, acc.at[colbuf.at[b]], add=True)

    idx_start(0, 0)

    def pair(p, _):
        for b in range(2):
            jj = 2 * p + 1 + b
            nb = (1 + b) % 2
            cb = b
            idx_start(jj, nb)
            idx_wait(cb)
            scat(cb)
        return 0

    lax.fori_loop(0, (NLOC - 2) // 2, pair, 0)
    idx_start(NLOC - 1, 1)
    idx_wait(0)
    scat(0)
    idx_wait(1)
    scat(1)
    plsc.subcore_barrier()

    pltpu.sync_copy(acc.at[pl.ds(s * DEG_PER_SUB, DEG_PER_SUB)],
                    out_hbm.at[c, pl.ds(s * DEG_PER_SUB, DEG_PER_SUB)])


@functools.lru_cache(maxsize=None)
def _deg2_kernel():
    return pl.kernel(
        _deg2_body,
        out_type=jax.ShapeDtypeStruct((NC, N1), jnp.float32),
        mesh=_sc_mesh(),
        scratch_types=[
            pltpu.VMEM((2, C), jnp.int32),
            pltpu.VMEM((C,), jnp.float32),
            pltpu.VMEM_SHARED((N1,), jnp.float32),
            pltpu.SemaphoreType.DMA((2,)),
        ],
    )


def _ppass_body(y_hbm, rc_hbm, zeros_hbm, out_hbm,
                idxbuf, rows, acc, semg, semi):
    c = lax.axis_index("c")
    s = lax.axis_index("s")
    wid = s * NC + c

    pltpu.sync_copy(zeros_hbm, acc.at[pl.ds(s * ROWS_PER_SUB, ROWS_PER_SUB)])
    plsc.subcore_barrier()

    def idx_start(jj, b):
        g = lax.min(wid + jj * NW, NCHUNK2 - 1)
        pltpu.async_copy(rc_hbm.at[g], idxbuf.at[b], semi.at[b])

    def idx_wait(b):
        pltpu.make_async_copy(rc_hbm.at[0], idxbuf.at[b], semi.at[b]).wait()

    def gather_start(b):
        pltpu.async_copy(y_hbm.at[idxbuf.at[b, 0]], rows.at[b], semg.at[b])

    def gather_wait(b):
        pltpu.make_async_copy(y_hbm.at[idxbuf.at[b, 0]], rows.at[b],
                              semg.at[b]).wait()

    def scat(b):
        pltpu.sync_copy(rows.at[b], acc.at[idxbuf.at[b, 1]], add=True)

    for j in range(NBUF - 1):
        idx_start(j, j)
    for j in range(NBUF - 1):
        idx_wait(j)
        gather_start(j)
    idx_start(NBUF - 1, NBUF - 1)

    TRIPS = NLOC // NBUF - 1

    def trip(p, _):
        for b in range(NBUF):
            b2 = (b + NBUF - 1) % NBUF
            jj = NBUF * p + b
            idx_wait(b2)
            gather_start(b2)
            gather_wait(b)
            scat(b)
            idx_start(jj + NBUF, b)
        return 0

    lax.fori_loop(0, TRIPS, trip, 0)
    idx_wait((NLOC - 1) % NBUF)
    gather_start((NLOC - 1) % NBUF)
    for j in range(NLOC - NBUF, NLOC):
        gather_wait(j % NBUF)
        scat(j % NBUF)
    plsc.subcore_barrier()

    pltpu.sync_copy(acc.at[pl.ds(s * ROWS_PER_SUB, ROWS_PER_SUB)],
                    out_hbm.at[c, pl.ds(s * ROWS_PER_SUB, ROWS_PER_SUB)])


@functools.lru_cache(maxsize=None)
def _ppass_kernel():
    return pl.kernel(
        _ppass_body,
        out_type=jax.ShapeDtypeStruct((NC, N1, D), jnp.float32),
        mesh=_sc_mesh(),
        scratch_types=[
            pltpu.VMEM((NBUF, 2, C), jnp.int32),
            pltpu.VMEM((NBUF, C, D), jnp.float32),
            pltpu.VMEM_SHARED((N1, D), jnp.float32),
            pltpu.SemaphoreType.DMA((NBUF,)),
            pltpu.SemaphoreType.DMA((NBUF,)),
        ],
    )


BN = 1000
GRID = N // BN


def _mlp_body(x_ref, wT_ref, b_ref, h_ref):
    h = jnp.dot(x_ref[...], wT_ref[...], precision=_HI,
                preferred_element_type=jnp.float32)
    h_ref[...] = jnp.maximum(h + b_ref[...], 0.0)


def _mlp(x, mlp_wT, mlp_b2):
    return pl.pallas_call(
        _mlp_body,
        grid=(GRID,),
        in_specs=[
            pl.BlockSpec((BN, D), lambda i: (i, 0)),
            pl.BlockSpec((D, D), lambda i: (0, 0)),
            pl.BlockSpec((1, D), lambda i: (0, 0)),
        ],
        out_specs=pl.BlockSpec((BN, D), lambda i: (i, 0)),
        out_shape=jax.ShapeDtypeStruct((N, D), jnp.float32),
    )(x, mlp_wT, mlp_b2)


def _scale0_body(deg_ref, h_ref, dinv_ref, y_ref):
    deg = deg_ref[0] + deg_ref[1]
    dinv = jnp.where(deg > 0, lax.rsqrt(jnp.maximum(deg, 1e-12)), 0.0)
    dinv_ref[...] = dinv
    y_ref[...] = h_ref[...] * dinv


def _scale0(deg_col, h):
    return pl.pallas_call(
        _scale0_body,
        grid=(GRID,),
        in_specs=[
            pl.BlockSpec((NC, BN, 1), lambda i: (0, i, 0)),
            pl.BlockSpec((BN, D), lambda i: (i, 0)),
        ],
        out_specs=[
            pl.BlockSpec((BN, 1), lambda i: (i, 0)),
            pl.BlockSpec((BN, D), lambda i: (i, 0)),
        ],
        out_shape=[
            jax.ShapeDtypeStruct((N, 1), jnp.float32),
            jax.ShapeDtypeStruct((N, D), jnp.float32),
        ],
    )(deg_col, h)


def _comb_body(part_ref, dinv_ref, zlast_ref, zprev_ref, z_ref, y_ref,
               *, ca, cb, cc):
    dinv = dinv_ref[...]
    s = (part_ref[0] + part_ref[1]) * dinv
    z = ca * s + cb * zlast_ref[...] + cc * zprev_ref[...]
    z_ref[...] = z
    y_ref[...] = z * dinv


def _comb(part, dinv_col, z_last, z_prev, ca, cb, cc):
    return pl.pallas_call(
        functools.partial(_comb_body, ca=ca, cb=cb, cc=cc),
        grid=(GRID,),
        in_specs=[
            pl.BlockSpec((NC, BN, D), lambda i: (0, i, 0)),
            pl.BlockSpec((BN, 1), lambda i: (i, 0)),
            pl.BlockSpec((BN, D), lambda i: (i, 0)),
            pl.BlockSpec((BN, D), lambda i: (i, 0)),
        ],
        out_specs=[
            pl.BlockSpec((BN, D), lambda i: (i, 0)),
            pl.BlockSpec((BN, D), lambda i: (i, 0)),
        ],
        out_shape=[
            jax.ShapeDtypeStruct((N, D), jnp.float32),
            jax.ShapeDtypeStruct((N, D), jnp.float32),
        ],
    )(part, dinv_col, z_last, z_prev)


def _q_body(zs_ref, w_ref, b_ref, q_ref):
    i = pl.program_id(0)

    @pl.when(i == 0)
    def _():
        q_ref[...] = jnp.zeros_like(q_ref)

    q_ref[...] += jnp.sum(zs_ref[...], axis=1)

    @pl.when(i == GRID - 1)
    def _():
        zbar = q_ref[...] / float(N)
        rows = [
            jnp.dot(zbar[k:k + 1, :], w_ref[k], precision=_HI,
                    preferred_element_type=jnp.float32) + b_ref[k:k + 1, :]
            for k in range(K + 1)
        ]
        q_ref[...] = jnp.concatenate(rows, axis=0)


def _q_kernel(zs, w, b):
    return pl.pallas_call(
        _q_body,
        grid=(GRID,),
        in_specs=[
            pl.BlockSpec((K + 1, BN, D), lambda i: (0, i, 0)),
            pl.BlockSpec((K + 1, D, D), lambda i: (0, 0, 0)),
            pl.BlockSpec((K + 1, D), lambda i: (0, 0)),
        ],
        out_specs=pl.BlockSpec((K + 1, D), lambda i: (0, 0)),
        out_shape=jax.ShapeDtypeStruct((K + 1, D), jnp.float32),
    )(zs, w, b)


def _final_body(zs_ref, w_ref, b_ref, q_ref, clsT_ref, clsb_ref,
                out_ref, zt_ref, alpha_ref):
    hs = [
        jnp.dot(zs_ref[k], w_ref[k], precision=_HI,
                preferred_element_type=jnp.float32) + b_ref[k:k + 1, :]
        for k in range(K + 1)
    ]
    scores = jnp.concatenate(
        [jnp.sum(hs[k] * q_ref[k:k + 1, :], axis=1, keepdims=True)
         for k in range(K + 1)], axis=1)
    scores = jnp.tanh(scores)
    m = jnp.max(scores, axis=1, keepdims=True)
    ex = jnp.exp(scores - m)
    alpha = ex / jnp.sum(ex, axis=1, keepdims=True)
    alpha_ref[...] = alpha
    zt = alpha[:, 0:1] * hs[0]
    for k in range(1, K + 1):
        zt = zt + alpha[:, k:k + 1] * hs[k]
    zt = jnp.maximum(zt, 0.0)
    zt_ref[...] = zt
    out_ref[...] = jnp.dot(zt, clsT_ref[...], precision=_HI,
                           preferred_element_type=jnp.float32) + clsb_ref[...]


def _final(zs, w, b, q, cls_wT, cls_b2):
    return pl.pallas_call(
        _final_body,
        grid=(GRID,),
        in_specs=[
            pl.BlockSpec((K + 1, BN, D), lambda i: (0, i, 0)),
            pl.BlockSpec((K + 1, D, D), lambda i: (0, 0, 0)),
            pl.BlockSpec((K + 1, D), lambda i: (0, 0)),
            pl.BlockSpec((K + 1, D), lambda i: (0, 0)),
            pl.BlockSpec((D, OUT), lambda i: (0, 0)),
            pl.BlockSpec((1, OUT), lambda i: (0, 0)),
        ],
        out_specs=[
            pl.BlockSpec((BN, OUT), lambda i: (i, 0)),
            pl.BlockSpec((BN, D), lambda i: (i, 0)),
            pl.BlockSpec((BN, K + 1), lambda i: (i, 0)),
        ],
        out_shape=[
            jax.ShapeDtypeStruct((N, OUT), jnp.float32),
            jax.ShapeDtypeStruct((N, D), jnp.float32),
            jax.ShapeDtypeStruct((N, K + 1), jnp.float32),
        ],
    )(zs, w, b, q, cls_wT, cls_b2)


def kernel(x, edge_index, mlp_w, mlp_b, W_weight, W_bias, cls_w, cls_b):
    row = edge_index[0].astype(jnp.int32)
    col = edge_index[1].astype(jnp.int32)

    pad = E2 - E
    padfill = jnp.full((pad,), PAD_ROW, jnp.int32)
    colp = jnp.concatenate([col, jnp.zeros((pad,), jnp.int32)])
    rowp = jnp.concatenate([row, padfill])
    rc = jnp.stack([colp.reshape(NCHUNK2, C), rowp.reshape(NCHUNK2, C)],
                   axis=1)
    col_deg = jnp.concatenate([col, padfill])

    ones_c = jnp.ones((C,), jnp.float32)
    zeros1 = jnp.zeros((DEG_PER_SUB,), jnp.float32)
    zeros2 = jnp.zeros((ROWS_PER_SUB, D), jnp.float32)

    h = _mlp(x, mlp_w.T, mlp_b.reshape(1, D))

    deg2 = _deg2_kernel()(col_deg, ones_c, zeros1)
    deg_col = deg2[:, :N].reshape(NC, N, 1)

    dinv_col, y = _scale0(deg_col, h)

    z_list = [h]
    part = _ppass_kernel()(y, rc, zeros2)
    z, y = _comb(part, dinv_col, h, h, _COEF2, _COEF1, 0.0)
    z_list.append(z)
    for k in range(2, K + 1):
        phi_k, phi_p, phi_pp = _phis(k)
        part = _ppass_kernel()(y, rc, zeros2)
        z, y = _comb(part, dinv_col, z_list[-1], z_list[-2],
                     phi_k, phi_p, -phi_pp)
        z_list.append(z)

    zs = jnp.stack(z_list, axis=0)
    q = _q_kernel(zs, W_weight, W_bias)
    out, zt, alpha = _final(zs, W_weight, W_bias, q, cls_w.T,
                            cls_b.reshape(1, OUT))
    return (out, zt, zs, alpha)


# speedup vs baseline: 10.1816x; 1.6111x over previous
import functools

import jax
import jax.numpy as jnp
from jax import lax
from jax.experimental import pallas as pl
from jax.experimental.pallas import tpu as pltpu
from jax.experimental.pallas import tpu_sc as plsc

N = 10000
E = 320000
D = 128
OUT = 64
K = 4
A = 1.0
B = 1.0

NC = 2
NS = 16
NW = NC * NS
C = 120
NCHUNK = -(-E // C)
NCHUNK2 = -(-NCHUNK // NW) * NW
E2 = NCHUNK2 * C
NLOC = NCHUNK2 // NW
assert NLOC % 3 == 0 and NLOC % 2 == 0
N1 = 10240
ROWS_PER_SUB = N1 // NS
DEG_PER_SUB = N1 // NS
PAD_ROW = N1 - 1

_HI = jax.lax.Precision.HIGHEST

_COEF1 = (A - B) / 2.0
_COEF2 = (A + B + 2.0) / 2.0


def _phis(k):
    phi_k = (2 * k + A + B) * (2 * k + A + B - 1) / (2 * k * (k + A + B))
    phi_p = ((2 * k + A + B - 1) * (A ** 2 - B ** 2)
             / (2 * k * (k + A + B) * (2 * k + A + B - 2)))
    phi_pp = ((k + A - 1) * (k + B - 1) * (2 * k + A + B)
              / (k * (k + A + B) * (2 * k + A + B - 2)))
    return phi_k, phi_p, phi_pp


@functools.lru_cache(maxsize=None)
def _sc_mesh():
    return plsc.VectorSubcoreMesh(core_axis_name="c", subcore_axis_name="s",
                                  num_cores=NC, num_subcores=NS)


def _deg2_body(col_hbm, ones_hbm, zeros1_hbm, out_hbm,
               colbuf, ones_v, acc, sem):
    c = lax.axis_index("c")
    s = lax.axis_index("s")
    wid = s * NC + c

    pltpu.sync_copy(zeros1_hbm.at[pl.ds(0, DEG_PER_SUB)],
                    acc.at[pl.ds(s * DEG_PER_SUB, DEG_PER_SUB)])
    pltpu.sync_copy(ones_hbm, ones_v)
    plsc.subcore_barrier()

    def idx_start(jj, b):
        base = (wid + jj * NW) * C
        pltpu.async_copy(col_hbm.at[pl.ds(base, C)], colbuf.at[b], sem.at[b])

    def idx_wait(b):
        pltpu.make_async_copy(col_hbm.at[pl.ds(0, C)], colbuf.at[b],
                              sem.at[b]).wait()

    def scat(b):
        pltpu.sync_copy(ones_v, acc.at[colbuf.at[b]], add=True)

    idx_start(0, 0)

    def pair(p, _):
        for b in range(2):
            jj = 2 * p + 1 + b
            nb = (1 + b) % 2
            cb = b
            idx_start(jj, nb)
            idx_wait(cb)
            scat(cb)
        return 0

    lax.fori_loop(0, (NLOC - 2) // 2, pair, 0)
    idx_start(NLOC - 1, 1)
    idx_wait(0)
    scat(0)
    idx_wait(1)
    scat(1)
    plsc.subcore_barrier()

    pltpu.sync_copy(acc.at[pl.ds(s * DEG_PER_SUB, DEG_PER_SUB)],
                    out_hbm.at[c, pl.ds(s * DEG_PER_SUB, DEG_PER_SUB)])


@functools.lru_cache(maxsize=None)
def _deg2_kernel():
    return pl.kernel(
        _deg2_body,
        out_type=jax.ShapeDtypeStruct((NC, N1), jnp.float32),
        mesh=_sc_mesh(),
        scratch_types=[
            pltpu.VMEM((2, C), jnp.int32),
            pltpu.VMEM((C,), jnp.float32),
            pltpu.VMEM_SHARED((N1,), jnp.float32),
            pltpu.SemaphoreType.DMA((2,)),
        ],
    )


def _ppass_body(y_hbm, rc_hbm, zeros_hbm, out_hbm,
                idxbuf, rows, acc, semg, semi):
    c = lax.axis_index("c")
    s = lax.axis_index("s")
    wid = s * NC + c

    pltpu.sync_copy(zeros_hbm, acc.at[pl.ds(s * ROWS_PER_SUB, ROWS_PER_SUB)])
    plsc.subcore_barrier()

    def idx_start(jj, b):
        g = lax.min(wid + jj * NW, NCHUNK2 - 1)
        pltpu.async_copy(rc_hbm.at[g], idxbuf.at[b], semi.at[b])

    def idx_wait(b):
        pltpu.make_async_copy(rc_hbm.at[0], idxbuf.at[b], semi.at[b]).wait()

    def gather_start(b):
        pltpu.async_copy(y_hbm.at[idxbuf.at[b, 0]], rows.at[b], semg.at[b])

    def gather_wait(b):
        pltpu.make_async_copy(y_hbm.at[idxbuf.at[b, 0]], rows.at[b],
                              semg.at[b]).wait()

    def scat(b):
        pltpu.sync_copy(rows.at[b], acc.at[idxbuf.at[b, 1]], add=True)

    for j in range(2):
        idx_start(j, j)
    for j in range(2):
        idx_wait(j)
        gather_start(j)
    idx_start(2, 2)

    TRIPS = NLOC // 3 - 1

    def trip(p, _):
        for b in range(3):
            b2 = (b + 2) % 3
            jj = 3 * p + b
            idx_wait(b2)
            gather_start(b2)
            gather_wait(b)
            scat(b)
            idx_start(jj + 3, b)
        return 0

    lax.fori_loop(0, TRIPS, trip, 0)
    idx_wait((NLOC - 1) % 3)
    gather_start((NLOC - 1) % 3)
    for j in range(NLOC - 3, NLOC):
        gather_wait(j % 3)
        scat(j % 3)
    plsc.subcore_barrier()

    pltpu.sync_copy(acc.at[pl.ds(s * ROWS_PER_SUB, ROWS_PER_SUB)],
                    out_hbm.at[c, pl.ds(s * ROWS_PER_SUB, ROWS_PER_SUB)])


@functools.lru_cache(maxsize=None)
def _ppass_kernel():
    return pl.kernel(
        _ppass_body,
        out_type=jax.ShapeDtypeStruct((NC, N1, D), jnp.float32),
        mesh=_sc_mesh(),
        scratch_types=[
            pltpu.VMEM((3, 2, C), jnp.int32),
            pltpu.VMEM((3, C, D), jnp.float32),
            pltpu.VMEM_SHARED((N1, D), jnp.float32),
            pltpu.SemaphoreType.DMA((3,)),
            pltpu.SemaphoreType.DMA((3,)),
        ],
    )


BN = 1000
GRID = N // BN


def _mlpscale_body(x_ref, wT_ref, b_ref, deg_ref,
                   h_ref, dinv_ref, y_ref, hsum_ref):
    i = pl.program_id(0)
    h = jnp.dot(x_ref[...], wT_ref[...], precision=_HI,
                preferred_element_type=jnp.float32)
    h = jnp.maximum(h + b_ref[...], 0.0)
    h_ref[...] = h
    deg = deg_ref[0] + deg_ref[1]
    dinv = jnp.where(deg > 0, lax.rsqrt(jnp.maximum(deg, 1e-12)), 0.0)
    dinv_ref[...] = dinv
    y_ref[...] = h * dinv

    @pl.when(i == 0)
    def _():
        hsum_ref[...] = jnp.zeros_like(hsum_ref)

    hsum_ref[...] += jnp.sum(h, axis=0, keepdims=True)


def _mlpscale(x, mlp_wT, mlp_b2, deg_col):
    return pl.pallas_call(
        _mlpscale_body,
        grid=(GRID,),
        in_specs=[
            pl.BlockSpec((BN, D), lambda i: (i, 0)),
            pl.BlockSpec((D, D), lambda i: (0, 0)),
            pl.BlockSpec((1, D), lambda i: (0, 0)),
            pl.BlockSpec((NC, BN, 1), lambda i: (0, i, 0)),
        ],
        out_specs=[
            pl.BlockSpec((BN, D), lambda i: (i, 0)),
            pl.BlockSpec((BN, 1), lambda i: (i, 0)),
            pl.BlockSpec((BN, D), lambda i: (i, 0)),
            pl.BlockSpec((1, D), lambda i: (0, 0)),
        ],
        out_shape=[
            jax.ShapeDtypeStruct((N, D), jnp.float32),
            jax.ShapeDtypeStruct((N, 1), jnp.float32),
            jax.ShapeDtypeStruct((N, D), jnp.float32),
            jax.ShapeDtypeStruct((1, D), jnp.float32),
        ],
    )(x, mlp_wT, mlp_b2, deg_col)


def _comb_body(part_ref, dinv_ref, zlast_ref, zprev_ref,
               z_ref, y_ref, zsum_ref, *, ca, cb, cc):
    i = pl.program_id(0)
    dinv = dinv_ref[...]
    s = (part_ref[0] + part_ref[1]) * dinv
    z = ca * s + cb * zlast_ref[...] + cc * zprev_ref[...]
    z_ref[...] = z
    y_ref[...] = z * dinv

    @pl.when(i == 0)
    def _():
        zsum_ref[...] = jnp.zeros_like(zsum_ref)

    zsum_ref[...] += jnp.sum(z, axis=0, keepdims=True)


def _comb(part, dinv_col, z_last, z_prev, ca, cb, cc):
    return pl.pallas_call(
        functools.partial(_comb_body, ca=ca, cb=cb, cc=cc),
        grid=(GRID,),
        in_specs=[
            pl.BlockSpec((NC, BN, D), lambda i: (0, i, 0)),
            pl.BlockSpec((BN, 1), lambda i: (i, 0)),
            pl.BlockSpec((BN, D), lambda i: (i, 0)),
            pl.BlockSpec((BN, D), lambda i: (i, 0)),
        ],
        out_specs=[
            pl.BlockSpec((BN, D), lambda i: (i, 0)),
            pl.BlockSpec((BN, D), lambda i: (i, 0)),
            pl.BlockSpec((1, D), lambda i: (0, 0)),
        ],
        out_shape=[
            jax.ShapeDtypeStruct((N, D), jnp.float32),
            jax.ShapeDtypeStruct((N, D), jnp.float32),
            jax.ShapeDtypeStruct((1, D), jnp.float32),
        ],
    )(part, dinv_col, z_last, z_prev)


def _final_body(zs_ref, w_ref, b_ref, zbar_ref, clsT_ref, clsb_ref,
                out_ref, zt_ref, alpha_ref):
    zbar = zbar_ref[...] / float(N)
    qs = [
        jnp.dot(zbar[k:k + 1, :], w_ref[k], precision=_HI,
                preferred_element_type=jnp.float32) + b_ref[k:k + 1, :]
        for k in range(K + 1)
    ]
    hs = [
        jnp.dot(zs_ref[k], w_ref[k], precision=_HI,
                preferred_element_type=jnp.float32) + b_ref[k:k + 1, :]
        for k in range(K + 1)
    ]
    scores = jnp.concatenate(
        [jnp.sum(hs[k] * qs[k], axis=1, keepdims=True)
         for k in range(K + 1)], axis=1)
    scores = jnp.tanh(scores)
    m = jnp.max(scores, axis=1, keepdims=True)
    ex = jnp.exp(scores - m)
    alpha = ex / jnp.sum(ex, axis=1, keepdims=True)
    alpha_ref[...] = alpha
    zt = alpha[:, 0:1] * hs[0]
    for k in range(1, K + 1):
        zt = zt + alpha[:, k:k + 1] * hs[k]
    zt = jnp.maximum(zt, 0.0)
    zt_ref[...] = zt
    out_ref[...] = jnp.dot(zt, clsT_ref[...], precision=_HI,
                           preferred_element_type=jnp.float32) + clsb_ref[...]


def _final(zs, w, b, q, cls_wT, cls_b2):
    return pl.pallas_call(
        _final_body,
        grid=(GRID,),
        in_specs=[
            pl.BlockSpec((K + 1, BN, D), lambda i: (0, i, 0)),
            pl.BlockSpec((K + 1, D, D), lambda i: (0, 0, 0)),
            pl.BlockSpec((K + 1, D), lambda i: (0, 0)),
            pl.BlockSpec((K + 1, D), lambda i: (0, 0)),
            pl.BlockSpec((D, OUT), lambda i: (0, 0)),
            pl.BlockSpec((1, OUT), lambda i: (0, 0)),
        ],
        out_specs=[
            pl.BlockSpec((BN, OUT), lambda i: (i, 0)),
            pl.BlockSpec((BN, D), lambda i: (i, 0)),
            pl.BlockSpec((BN, K + 1), lambda i: (i, 0)),
        ],
        out_shape=[
            jax.ShapeDtypeStruct((N, OUT), jnp.float32),
            jax.ShapeDtypeStruct((N, D), jnp.float32),
            jax.ShapeDtypeStruct((N, K + 1), jnp.float32),
        ],
    )(zs, w, b, q, cls_wT, cls_b2)


def kernel(x, edge_index, mlp_w, mlp_b, W_weight, W_bias, cls_w, cls_b):
    row = edge_index[0].astype(jnp.int32)
    col = edge_index[1].astype(jnp.int32)

    pad = E2 - E
    padfill = jnp.full((pad,), PAD_ROW, jnp.int32)
    colp = jnp.concatenate([col, jnp.zeros((pad,), jnp.int32)])
    rowp = jnp.concatenate([row, padfill])
    rc = jnp.stack([colp.reshape(NCHUNK2, C), rowp.reshape(NCHUNK2, C)],
                   axis=1)
    col_deg = jnp.concatenate([col, padfill])

    ones_c = jnp.ones((C,), jnp.float32)
    zeros1 = jnp.zeros((DEG_PER_SUB,), jnp.float32)
    zeros2 = jnp.zeros((ROWS_PER_SUB, D), jnp.float32)

    deg2 = _deg2_kernel()(col_deg, ones_c, zeros1)
    deg_col = deg2[:, :N].reshape(NC, N, 1)

    h, dinv_col, y, hsum = _mlpscale(x, mlp_w.T, mlp_b.reshape(1, D), deg_col)

    z_list = [h]
    zsums = [hsum]
    part = _ppass_kernel()(y, rc, zeros2)
    z, y, zsum = _comb(part, dinv_col, h, h, _COEF2, _COEF1, 0.0)
    z_list.append(z)
    zsums.append(zsum)
    for k in range(2, K + 1):
        phi_k, phi_p, phi_pp = _phis(k)
        part = _ppass_kernel()(y, rc, zeros2)
        z, y, zsum = _comb(part, dinv_col, z_list[-1], z_list[-2],
                           phi_k, phi_p, -phi_pp)
        z_list.append(z)
        zsums.append(zsum)

    zs = jnp.stack(z_list, axis=0)
    zbar = jnp.concatenate(zsums, axis=0)
    out, zt, alpha = _final(zs, W_weight, W_bias, zbar, cls_w.T,
                            cls_b.reshape(1, OUT))
    return (out, zt, zs, alpha)


# speedup vs baseline: 10.2350x; 1.0052x over previous
import functools

import jax
import jax.numpy as jnp
from jax import lax
from jax.experimental import pallas as pl
from jax.experimental.pallas import tpu as pltpu
from jax.experimental.pallas import tpu_sc as plsc

N = 10000
E = 320000
D = 128
OUT = 64
K = 4
A = 1.0
B = 1.0

NC = 2
NS = 16
NW = NC * NS
C = 120
NCHUNK = -(-E // C)
NCHUNK2 = -(-NCHUNK // NW) * NW
E2 = NCHUNK2 * C
NLOC = NCHUNK2 // NW
assert NLOC % 3 == 0 and NLOC % 2 == 0
N1 = 10240
ROWS_PER_SUB = N1 // NS
DEG_PER_SUB = N1 // NS
PAD_ROW = N1 - 1

_HI = jax.lax.Precision.HIGHEST

_COEF1 = (A - B) / 2.0
_COEF2 = (A + B + 2.0) / 2.0


def _phis(k):
    phi_k = (2 * k + A + B) * (2 * k + A + B - 1) / (2 * k * (k + A + B))
    phi_p = ((2 * k + A + B - 1) * (A ** 2 - B ** 2)
             / (2 * k * (k + A + B) * (2 * k + A + B - 2)))
    phi_pp = ((k + A - 1) * (k + B - 1) * (2 * k + A + B)
              / (k * (k + A + B) * (2 * k + A + B - 2)))
    return phi_k, phi_p, phi_pp


@functools.lru_cache(maxsize=None)
def _sc_mesh():
    return plsc.VectorSubcoreMesh(core_axis_name="c", subcore_axis_name="s",
                                  num_cores=NC, num_subcores=NS)


def _deg2_body(col_hbm, ones_hbm, zeros1_hbm, out_hbm,
               colbuf, ones_v, acc, sem):
    c = lax.axis_index("c")
    s = lax.axis_index("s")
    wid = s * NC + c

    pltpu.sync_copy(zeros1_hbm.at[pl.ds(0, DEG_PER_SUB)],
                    acc.at[pl.ds(s * DEG_PER_SUB, DEG_PER_SUB)])
    pltpu.sync_copy(ones_hbm, ones_v)
    plsc.subcore_barrier()

    def idx_start(jj, b):
        base = (wid + jj * NW) * C
        pltpu.async_copy(col_hbm.at[pl.ds(base, C)], colbuf.at[b], sem.at[b])

    def idx_wait(b):
        pltpu.make_async_copy(col_hbm.at[pl.ds(0, C)], colbuf.at[b],
                              sem.at[b]).wait()

    def scat(b):
        pltpu.sync_copy(ones_v, acc.at[colbuf.at[b]], add=True)

    idx_start(0, 0)

    def pair(p, _):
        for b in range(2):
            jj = 2 * p + 1 + b
            nb = (1 + b) % 2
            cb = b
            idx_start(jj, nb)
            idx_wait(cb)
            scat(cb)
        return 0

    lax.fori_loop(0, (NLOC - 2) // 2, pair, 0)
    idx_start(NLOC - 1, 1)
    idx_wait(0)
    scat(0)
    idx_wait(1)
    scat(1)
    plsc.subcore_barrier()

    pltpu.sync_copy(acc.at[pl.ds(s * DEG_PER_SUB, DEG_PER_SUB)],
                    out_hbm.at[c, pl.ds(s * DEG_PER_SUB, DEG_PER_SUB)])


@functools.lru_cache(maxsize=None)
def _deg2_kernel():
    return pl.kernel(
        _deg2_body,
        out_type=jax.ShapeDtypeStruct((NC, N1), jnp.float32),
        mesh=_sc_mesh(),
        scratch_types=[
            pltpu.VMEM((2, C), jnp.int32),
            pltpu.VMEM((C,), jnp.float32),
            pltpu.VMEM_SHARED((N1,), jnp.float32),
            pltpu.SemaphoreType.DMA((2,)),
        ],
    )


def _ppass_body(y_hbm, rc_hbm, zeros_hbm, out_hbm,
                idxbuf, rows, acc, semg, semi):
    c = lax.axis_index("c")
    s = lax.axis_index("s")
    wid = s * NC + c

    def idx_start(jj, b):
        g = lax.min(wid + jj * NW, NCHUNK2 - 1)
        pltpu.async_copy(rc_hbm.at[g], idxbuf.at[b], semi.at[b])

    def idx_wait(b):
        pltpu.make_async_copy(rc_hbm.at[0], idxbuf.at[b], semi.at[b]).wait()

    def gather_start(b):
        pltpu.async_copy(y_hbm.at[idxbuf.at[b, 0]], rows.at[b], semg.at[b])

    def gather_wait(b):
        pltpu.make_async_copy(y_hbm.at[idxbuf.at[b, 0]], rows.at[b],
                              semg.at[b]).wait()

    def scat(b):
        pltpu.sync_copy(rows.at[b], acc.at[idxbuf.at[b, 1]], add=True)

    for j in range(2):
        idx_start(j, j)
    for j in range(2):
        idx_wait(j)
        gather_start(j)
    idx_start(2, 2)
    pltpu.sync_copy(zeros_hbm, acc.at[pl.ds(s * ROWS_PER_SUB, ROWS_PER_SUB)])
    plsc.subcore_barrier()

    TRIPS = NLOC // 3 - 1

    def trip(p, _):
        for b in range(3):
            b2 = (b + 2) % 3
            jj = 3 * p + b
            idx_wait(b2)
            gather_start(b2)
            gather_wait(b)
            scat(b)
            idx_start(jj + 3, b)
        return 0

    lax.fori_loop(0, TRIPS, trip, 0)
    idx_wait((NLOC - 1) % 3)
    gather_start((NLOC - 1) % 3)
    for j in range(NLOC - 3, NLOC):
        gather_wait(j % 3)
        scat(j % 3)
    plsc.subcore_barrier()

    pltpu.sync_copy(acc.at[pl.ds(s * ROWS_PER_SUB, ROWS_PER_SUB)],
                    out_hbm.at[c, pl.ds(s * ROWS_PER_SUB, ROWS_PER_SUB)])


@functools.lru_cache(maxsize=None)
def _ppass_kernel():
    return pl.kernel(
        _ppass_body,
        out_type=jax.ShapeDtypeStruct((NC, N1, D), jnp.float32),
        mesh=_sc_mesh(),
        scratch_types=[
            pltpu.VMEM((3, 2, C), jnp.int32),
            pltpu.VMEM((3, C, D), jnp.float32),
            pltpu.VMEM_SHARED((N1, D), jnp.float32),
            pltpu.SemaphoreType.DMA((3,)),
            pltpu.SemaphoreType.DMA((3,)),
        ],
    )


BN = 1000
GRID = N // BN


def _mlpscale_body(x_ref, wT_ref, b_ref, deg_ref,
                   h_ref, dinv_ref, y_ref, hsum_ref):
    i = pl.program_id(0)
    h = jnp.dot(x_ref[...], wT_ref[...], precision=_HI,
                preferred_element_type=jnp.float32)
    h = jnp.maximum(h + b_ref[...], 0.0)
    h_ref[...] = h
    deg = deg_ref[0] + deg_ref[1]
    dinv = jnp.where(deg > 0, lax.rsqrt(jnp.maximum(deg, 1e-12)), 0.0)
    dinv_ref[...] = dinv
    y_ref[...] = h * dinv

    @pl.when(i == 0)
    def _():
        hsum_ref[...] = jnp.zeros_like(hsum_ref)

    hsum_ref[...] += jnp.sum(h, axis=0, keepdims=True)


def _mlpscale(x, mlp_wT, mlp_b2, deg_col):
    return pl.pallas_call(
        _mlpscale_body,
        grid=(GRID,),
        in_specs=[
            pl.BlockSpec((BN, D), lambda i: (i, 0)),
            pl.BlockSpec((D, D), lambda i: (0, 0)),
            pl.BlockSpec((1, D), lambda i: (0, 0)),
            pl.BlockSpec((NC, BN, 1), lambda i: (0, i, 0)),
        ],
        out_specs=[
            pl.BlockSpec((BN, D), lambda i: (i, 0)),
            pl.BlockSpec((BN, 1), lambda i: (i, 0)),
            pl.BlockSpec((BN, D), lambda i: (i, 0)),
            pl.BlockSpec((1, D), lambda i: (0, 0)),
        ],
        out_shape=[
            jax.ShapeDtypeStruct((N, D), jnp.float32),
            jax.ShapeDtypeStruct((N, 1), jnp.float32),
            jax.ShapeDtypeStruct((N, D), jnp.float32),
            jax.ShapeDtypeStruct((1, D), jnp.float32),
        ],
    )(x, mlp_wT, mlp_b2, deg_col)


def _comb_body(part_ref, dinv_ref, zlast_ref, zprev_ref,
               z_ref, y_ref, zsum_ref, *, ca, cb, cc):
    i = pl.program_id(0)
    dinv = dinv_ref[...]
    s = (part_ref[0] + part_ref[1]) * dinv
    z = ca * s + cb * zlast_ref[...] + cc * zprev_ref[...]
    z_ref[...] = z
    if y_ref is not None:
        y_ref[...] = z * dinv

    @pl.when(i == 0)
    def _():
        zsum_ref[...] = jnp.zeros_like(zsum_ref)

    zsum_ref[...] += jnp.sum(z, axis=0, keepdims=True)


def _comb(part, dinv_col, z_last, z_prev, ca, cb, cc, with_y=True):
    out_specs = [
        pl.BlockSpec((BN, D), lambda i: (i, 0)),
        pl.BlockSpec((BN, D), lambda i: (i, 0)),
        pl.BlockSpec((1, D), lambda i: (0, 0)),
    ]
    out_shape = [
        jax.ShapeDtypeStruct((N, D), jnp.float32),
        jax.ShapeDtypeStruct((N, D), jnp.float32),
        jax.ShapeDtypeStruct((1, D), jnp.float32),
    ]
    body = functools.partial(_comb_body, ca=ca, cb=cb, cc=cc)
    if not with_y:
        del out_specs[1], out_shape[1]
        full = body

        def body(part_ref, dinv_ref, zlast_ref, zprev_ref, z_ref, zsum_ref):
            full(part_ref, dinv_ref, zlast_ref, zprev_ref, z_ref, None,
                 zsum_ref)

    res = pl.pallas_call(
        body,
        grid=(GRID,),
        in_specs=[
            pl.BlockSpec((NC, BN, D), lambda i: (0, i, 0)),
            pl.BlockSpec((BN, 1), lambda i: (i, 0)),
            pl.BlockSpec((BN, D), lambda i: (i, 0)),
            pl.BlockSpec((BN, D), lambda i: (i, 0)),
        ],
        out_specs=out_specs,
        out_shape=out_shape,
    )(part, dinv_col, z_last, z_prev)
    if not with_y:
        return res[0], None, res[1]
    return res


def _final_body(zs_ref, w_ref, b_ref, zbar_ref, clsT_ref, clsb_ref,
                out_ref, zt_ref, alpha_ref):
    zbar = zbar_ref[...] / float(N)
    qs = [
        jnp.dot(zbar[k:k + 1, :], w_ref[k], precision=_HI,
                preferred_element_type=jnp.float32) + b_ref[k:k + 1, :]
        for k in range(K + 1)
    ]
    hs = [
        jnp.dot(zs_ref[k], w_ref[k], precision=_HI,
                preferred_element_type=jnp.float32) + b_ref[k:k + 1, :]
        for k in range(K + 1)
    ]
    scores = jnp.concatenate(
        [jnp.sum(hs[k] * qs[k], axis=1, keepdims=True)
         for k in range(K + 1)], axis=1)
    scores = jnp.tanh(scores)
    m = jnp.max(scores, axis=1, keepdims=True)
    ex = jnp.exp(scores - m)
    alpha = ex / jnp.sum(ex, axis=1, keepdims=True)
    alpha_ref[...] = alpha
    zt = alpha[:, 0:1] * hs[0]
    for k in range(1, K + 1):
        zt = zt + alpha[:, k:k + 1] * hs[k]
    zt = jnp.maximum(zt, 0.0)
    zt_ref[...] = zt
    out_ref[...] = jnp.dot(zt, clsT_ref[...], precision=_HI,
                           preferred_element_type=jnp.float32) + clsb_ref[...]


def _final(zs, w, b, q, cls_wT, cls_b2):
    return pl.pallas_call(
        _final_body,
        grid=(GRID,),
        in_specs=[
            pl.BlockSpec((K + 1, BN, D), lambda i: (0, i, 0)),
            pl.BlockSpec((K + 1, D, D), lambda i: (0, 0, 0)),
            pl.BlockSpec((K + 1, D), lambda i: (0, 0)),
            pl.BlockSpec((K + 1, D), lambda i: (0, 0)),
            pl.BlockSpec((D, OUT), lambda i: (0, 0)),
            pl.BlockSpec((1, OUT), lambda i: (0, 0)),
        ],
        out_specs=[
            pl.BlockSpec((BN, OUT), lambda i: (i, 0)),
            pl.BlockSpec((BN, D), lambda i: (i, 0)),
            pl.BlockSpec((BN, K + 1), lambda i: (i, 0)),
        ],
        out_shape=[
            jax.ShapeDtypeStruct((N, OUT), jnp.float32),
            jax.ShapeDtypeStruct((N, D), jnp.float32),
            jax.ShapeDtypeStruct((N, K + 1), jnp.float32),
        ],
    )(zs, w, b, q, cls_wT, cls_b2)


def kernel(x, edge_index, mlp_w, mlp_b, W_weight, W_bias, cls_w, cls_b):
    row = edge_index[0].astype(jnp.int32)
    col = edge_index[1].astype(jnp.int32)

    pad = E2 - E
    padfill = jnp.full((pad,), PAD_ROW, jnp.int32)
    colp = jnp.concatenate([col, jnp.zeros((pad,), jnp.int32)])
    rowp = jnp.concatenate([row, padfill])
    rc = jnp.stack([colp.reshape(NCHUNK2, C), rowp.reshape(NCHUNK2, C)],
                   axis=1)
    col_deg = jnp.concatenate([col, padfill])

    ones_c = jnp.ones((C,), jnp.float32)
    zeros1 = jnp.zeros((DEG_PER_SUB,), jnp.float32)
    zeros2 = jnp.zeros((ROWS_PER_SUB, D), jnp.float32)

    deg2 = _deg2_kernel()(col_deg, ones_c, zeros1)
    deg_col = deg2[:, :N].reshape(NC, N, 1)

    h, dinv_col, y, hsum = _mlpscale(x, mlp_w.T, mlp_b.reshape(1, D), deg_col)

    z_list = [h]
    zsums = [hsum]
    part = _ppass_kernel()(y, rc, zeros2)
    z, y, zsum = _comb(part, dinv_col, h, h, _COEF2, _COEF1, 0.0)
    z_list.append(z)
    zsums.append(zsum)
    for k in range(2, K + 1):
        phi_k, phi_p, phi_pp = _phis(k)
        part = _ppass_kernel()(y, rc, zeros2)
        z, y, zsum = _comb(part, dinv_col, z_list[-1], z_list[-2],
                           phi_k, phi_p, -phi_pp, with_y=(k < K))
        z_list.append(z)
        zsums.append(zsum)

    zs = jnp.stack(z_list, axis=0)
    zbar = jnp.concatenate(zsums, axis=0)
    out, zt, alpha = _final(zs, W_weight, W_bias, zbar, cls_w.T,
                            cls_b.reshape(1, OUT))
    return (out, zt, zs, alpha)


# speedup vs baseline: 10.5986x; 1.0355x over previous
import functools

import jax
import jax.numpy as jnp
from jax import lax
from jax.experimental import pallas as pl
from jax.experimental.pallas import tpu as pltpu
from jax.experimental.pallas import tpu_sc as plsc

N = 10000
E = 320000
D = 128
OUT = 64
K = 4
A = 1.0
B = 1.0

NC = 2
NS = 16
NW = NC * NS
C = 120
NCHUNK = -(-E // C)
NCHUNK2 = -(-NCHUNK // NW) * NW
E2 = NCHUNK2 * C
NLOC = NCHUNK2 // NW
assert NLOC % 3 == 0 and NLOC % 2 == 0
N1 = 10240
ROWS_PER_SUB = N1 // NS
DEG_PER_SUB = N1 // NS
PAD_ROW = N1 - 1

_HI = jax.lax.Precision.HIGHEST

_COEF1 = (A - B) / 2.0
_COEF2 = (A + B + 2.0) / 2.0


def _phis(k):
    phi_k = (2 * k + A + B) * (2 * k + A + B - 1) / (2 * k * (k + A + B))
    phi_p = ((2 * k + A + B - 1) * (A ** 2 - B ** 2)
             / (2 * k * (k + A + B) * (2 * k + A + B - 2)))
    phi_pp = ((k + A - 1) * (k + B - 1) * (2 * k + A + B)
              / (k * (k + A + B) * (2 * k + A + B - 2)))
    return phi_k, phi_p, phi_pp


@functools.lru_cache(maxsize=None)
def _sc_mesh():
    return plsc.VectorSubcoreMesh(core_axis_name="c", subcore_axis_name="s",
                                  num_cores=NC, num_subcores=NS)


def _deg2_body(col_hbm, ones_hbm, zeros1_hbm, out_hbm,
               colbuf, ones_v, acc, sem):
    c = lax.axis_index("c")
    s = lax.axis_index("s")
    wid = s * NC + c

    pltpu.sync_copy(zeros1_hbm.at[pl.ds(0, DEG_PER_SUB)],
                    acc.at[pl.ds(s * DEG_PER_SUB, DEG_PER_SUB)])
    pltpu.sync_copy(ones_hbm, ones_v)
    plsc.subcore_barrier()

    def idx_start(jj, b):
        base = (wid + jj * NW) * C
        pltpu.async_copy(col_hbm.at[pl.ds(base, C)], colbuf.at[b], sem.at[b])

    def idx_wait(b):
        pltpu.make_async_copy(col_hbm.at[pl.ds(0, C)], colbuf.at[b],
                              sem.at[b]).wait()

    def scat(b):
        pltpu.sync_copy(ones_v, acc.at[colbuf.at[b]], add=True)

    idx_start(0, 0)

    def pair(p, _):
        for b in range(2):
            jj = 2 * p + 1 + b
            nb = (1 + b) % 2
            cb = b
            idx_start(jj, nb)
            idx_wait(cb)
            scat(cb)
        return 0

    lax.fori_loop(0, (NLOC - 2) // 2, pair, 0)
    idx_start(NLOC - 1, 1)
    idx_wait(0)
    scat(0)
    idx_wait(1)
    scat(1)
    plsc.subcore_barrier()

    pltpu.sync_copy(acc.at[pl.ds(s * DEG_PER_SUB, DEG_PER_SUB)],
                    out_hbm.at[c, pl.ds(s * DEG_PER_SUB, DEG_PER_SUB)])


@functools.lru_cache(maxsize=None)
def _deg2_kernel():
    return pl.kernel(
        _deg2_body,
        out_type=jax.ShapeDtypeStruct((NC, N1), jnp.float32),
        mesh=_sc_mesh(),
        scratch_types=[
            pltpu.VMEM((2, C), jnp.int32),
            pltpu.VMEM((C,), jnp.float32),
            pltpu.VMEM_SHARED((N1,), jnp.float32),
            pltpu.SemaphoreType.DMA((2,)),
        ],
    )


def _ppass_body(y_hbm, rc_hbm, zeros_hbm, out_hbm,
                idxbuf, rows, acc, semg, semi):
    c = lax.axis_index("c")
    s = lax.axis_index("s")
    wid = s * NC + c

    def idx_start(jj, b):
        g = lax.min(wid + jj * NW, NCHUNK2 - 1)
        pltpu.async_copy(rc_hbm.at[g], idxbuf.at[b], semi.at[b])

    def idx_wait(b):
        pltpu.make_async_copy(rc_hbm.at[0], idxbuf.at[b], semi.at[b]).wait()

    def gather_start(b):
        pltpu.async_copy(y_hbm.at[idxbuf.at[b, 0]], rows.at[b], semg.at[b])

    def gather_wait(b):
        pltpu.make_async_copy(y_hbm.at[idxbuf.at[b, 0]], rows.at[b],
                              semg.at[b]).wait()

    def scat(b):
        pltpu.sync_copy(rows.at[b], acc.at[idxbuf.at[b, 1]], add=True)

    for j in range(2):
        idx_start(j, j)
    for j in range(2):
        idx_wait(j)
        gather_start(j)
    idx_start(2, 2)
    pltpu.sync_copy(zeros_hbm, acc.at[pl.ds(s * ROWS_PER_SUB, ROWS_PER_SUB)])
    plsc.subcore_barrier()

    TRIPS = NLOC // 3 - 1

    def trip(p, _):
        for b in range(3):
            b2 = (b + 2) % 3
            jj = 3 * p + b
            idx_wait(b2)
            gather_start(b2)
            gather_wait(b)
            scat(b)
            idx_start(jj + 3, b)
        return 0

    lax.fori_loop(0, TRIPS, trip, 0)
    idx_wait((NLOC - 1) % 3)
    gather_start((NLOC - 1) % 3)
    for j in range(NLOC - 3, NLOC):
        gather_wait(j % 3)
        scat(j % 3)
    plsc.subcore_barrier()

    pltpu.sync_copy(acc.at[pl.ds(s * ROWS_PER_SUB, ROWS_PER_SUB)],
                    out_hbm.at[c, pl.ds(s * ROWS_PER_SUB, ROWS_PER_SUB)])


@functools.lru_cache(maxsize=None)
def _ppass_kernel():
    return pl.kernel(
        _ppass_body,
        out_type=jax.ShapeDtypeStruct((NC, N1, D), jnp.float32),
        mesh=_sc_mesh(),
        scratch_types=[
            pltpu.VMEM((3, 2, C), jnp.int32),
            pltpu.VMEM((3, C, D), jnp.float32),
            pltpu.VMEM_SHARED((N1, D), jnp.float32),
            pltpu.SemaphoreType.DMA((3,)),
            pltpu.SemaphoreType.DMA((3,)),
        ],
    )


BN = 2000
GRID = N // BN


def _mlpscale_body(x_ref, wT_ref, b_ref, deg_ref,
                   h_ref, dinv_ref, y_ref, hsum_ref):
    i = pl.program_id(0)
    h = jnp.dot(x_ref[...], wT_ref[...], precision=_HI,
                preferred_element_type=jnp.float32)
    h = jnp.maximum(h + b_ref[...], 0.0)
    h_ref[...] = h
    deg = deg_ref[0] + deg_ref[1]
    dinv = jnp.where(deg > 0, lax.rsqrt(jnp.maximum(deg, 1e-12)), 0.0)
    dinv_ref[...] = dinv
    y_ref[...] = h * dinv

    @pl.when(i == 0)
    def _():
        hsum_ref[...] = jnp.zeros_like(hsum_ref)

    hsum_ref[...] += jnp.sum(h, axis=0, keepdims=True)


def _mlpscale(x, mlp_wT, mlp_b2, deg_col):
    return pl.pallas_call(
        _mlpscale_body,
        grid=(GRID,),
        in_specs=[
            pl.BlockSpec((BN, D), lambda i: (i, 0)),
            pl.BlockSpec((D, D), lambda i: (0, 0)),
            pl.BlockSpec((1, D), lambda i: (0, 0)),
            pl.BlockSpec((NC, BN, 1), lambda i: (0, i, 0)),
        ],
        out_specs=[
            pl.BlockSpec((BN, D), lambda i: (i, 0)),
            pl.BlockSpec((BN, 1), lambda i: (i, 0)),
            pl.BlockSpec((BN, D), lambda i: (i, 0)),
            pl.BlockSpec((1, D), lambda i: (0, 0)),
        ],
        out_shape=[
            jax.ShapeDtypeStruct((N, D), jnp.float32),
            jax.ShapeDtypeStruct((N, 1), jnp.float32),
            jax.ShapeDtypeStruct((N, D), jnp.float32),
            jax.ShapeDtypeStruct((1, D), jnp.float32),
        ],
    )(x, mlp_wT, mlp_b2, deg_col)


def _comb_body(part_ref, dinv_ref, zlast_ref, zprev_ref,
               z_ref, y_ref, zsum_ref, *, ca, cb, cc):
    i = pl.program_id(0)
    dinv = dinv_ref[...]
    s = (part_ref[0] + part_ref[1]) * dinv
    z = ca * s + cb * zlast_ref[...] + cc * zprev_ref[...]
    z_ref[...] = z
    if y_ref is not None:
        y_ref[...] = z * dinv

    @pl.when(i == 0)
    def _():
        zsum_ref[...] = jnp.zeros_like(zsum_ref)

    zsum_ref[...] += jnp.sum(z, axis=0, keepdims=True)


def _comb(part, dinv_col, z_last, z_prev, ca, cb, cc, with_y=True):
    out_specs = [
        pl.BlockSpec((BN, D), lambda i: (i, 0)),
        pl.BlockSpec((BN, D), lambda i: (i, 0)),
        pl.BlockSpec((1, D), lambda i: (0, 0)),
    ]
    out_shape = [
        jax.ShapeDtypeStruct((N, D), jnp.float32),
        jax.ShapeDtypeStruct((N, D), jnp.float32),
        jax.ShapeDtypeStruct((1, D), jnp.float32),
    ]
    body = functools.partial(_comb_body, ca=ca, cb=cb, cc=cc)
    if not with_y:
        del out_specs[1], out_shape[1]
        full = body

        def body(part_ref, dinv_ref, zlast_ref, zprev_ref, z_ref, zsum_ref):
            full(part_ref, dinv_ref, zlast_ref, zprev_ref, z_ref, None,
                 zsum_ref)

    res = pl.pallas_call(
        body,
        grid=(GRID,),
        in_specs=[
            pl.BlockSpec((NC, BN, D), lambda i: (0, i, 0)),
            pl.BlockSpec((BN, 1), lambda i: (i, 0)),
            pl.BlockSpec((BN, D), lambda i: (i, 0)),
            pl.BlockSpec((BN, D), lambda i: (i, 0)),
        ],
        out_specs=out_specs,
        out_shape=out_shape,
    )(part, dinv_col, z_last, z_prev)
    if not with_y:
        return res[0], None, res[1]
    return res


def _final_body(zs_ref, w_ref, b_ref, zbar_ref, clsT_ref, clsb_ref,
                out_ref, zt_ref, alpha_ref):
    zbar = zbar_ref[...] / float(N)
    qs = [
        jnp.dot(zbar[k:k + 1, :], w_ref[k], precision=_HI,
                preferred_element_type=jnp.float32) + b_ref[k:k + 1, :]
        for k in range(K + 1)
    ]
    hs = [
        jnp.dot(zs_ref[k], w_ref[k], precision=_HI,
                preferred_element_type=jnp.float32) + b_ref[k:k + 1, :]
        for k in range(K + 1)
    ]
    scores = jnp.concatenate(
        [jnp.sum(hs[k] * qs[k], axis=1, keepdims=True)
         for k in range(K + 1)], axis=1)
    scores = jnp.tanh(scores)
    m = jnp.max(scores, axis=1, keepdims=True)
    ex = jnp.exp(scores - m)
    alpha = ex / jnp.sum(ex, axis=1, keepdims=True)
    alpha_ref[...] = alpha
    zt = alpha[:, 0:1] * hs[0]
    for k in range(1, K + 1):
        zt = zt + alpha[:, k:k + 1] * hs[k]
    zt = jnp.maximum(zt, 0.0)
    zt_ref[...] = zt
    out_ref[...] = jnp.dot(zt, clsT_ref[...], precision=_HI,
                           preferred_element_type=jnp.float32) + clsb_ref[...]


def _final(zs, w, b, q, cls_wT, cls_b2):
    return pl.pallas_call(
        _final_body,
        grid=(GRID,),
        in_specs=[
            pl.BlockSpec((K + 1, BN, D), lambda i: (0, i, 0)),
            pl.BlockSpec((K + 1, D, D), lambda i: (0, 0, 0)),
            pl.BlockSpec((K + 1, D), lambda i: (0, 0)),
            pl.BlockSpec((K + 1, D), lambda i: (0, 0)),
            pl.BlockSpec((D, OUT), lambda i: (0, 0)),
            pl.BlockSpec((1, OUT), lambda i: (0, 0)),
        ],
        out_specs=[
            pl.BlockSpec((BN, OUT), lambda i: (i, 0)),
            pl.BlockSpec((BN, D), lambda i: (i, 0)),
            pl.BlockSpec((BN, K + 1), lambda i: (i, 0)),
        ],
        out_shape=[
            jax.ShapeDtypeStruct((N, OUT), jnp.float32),
            jax.ShapeDtypeStruct((N, D), jnp.float32),
            jax.ShapeDtypeStruct((N, K + 1), jnp.float32),
        ],
    )(zs, w, b, q, cls_wT, cls_b2)


def kernel(x, edge_index, mlp_w, mlp_b, W_weight, W_bias, cls_w, cls_b):
    row = edge_index[0].astype(jnp.int32)
    col = edge_index[1].astype(jnp.int32)

    pad = E2 - E
    padfill = jnp.full((pad,), PAD_ROW, jnp.int32)
    colp = jnp.concatenate([col, jnp.zeros((pad,), jnp.int32)])
    rowp = jnp.concatenate([row, padfill])
    rc = jnp.stack([colp.reshape(NCHUNK2, C), rowp.reshape(NCHUNK2, C)],
                   axis=1)
    col_deg = jnp.concatenate([col, padfill])

    ones_c = jnp.ones((C,), jnp.float32)
    zeros1 = jnp.zeros((DEG_PER_SUB,), jnp.float32)
    zeros2 = jnp.zeros((ROWS_PER_SUB, D), jnp.float32)

    deg2 = _deg2_kernel()(col_deg, ones_c, zeros1)
    deg_col = deg2[:, :N].reshape(NC, N, 1)

    h, dinv_col, y, hsum = _mlpscale(x, mlp_w.T, mlp_b.reshape(1, D), deg_col)

    z_list = [h]
    zsums = [hsum]
    part = _ppass_kernel()(y, rc, zeros2)
    z, y, zsum = _comb(part, dinv_col, h, h, _COEF2, _COEF1, 0.0)
    z_list.append(z)
    zsums.append(zsum)
    for k in range(2, K + 1):
        phi_k, phi_p, phi_pp = _phis(k)
        part = _ppass_kernel()(y, rc, zeros2)
        z, y, zsum = _comb(part, dinv_col, z_list[-1], z_list[-2],
                           phi_k, phi_p, -phi_pp, with_y=(k < K))
        z_list.append(z)
        zsums.append(zsum)

    zs = jnp.stack(z_list, axis=0)
    zbar = jnp.concatenate(zsums, axis=0)
    out, zt, alpha = _final(zs, W_weight, W_bias, zbar, cls_w.T,
                            cls_b.reshape(1, OUT))
    return (out, zt, zs, alpha)
